# Initial kernel scaffold; baseline (speedup 1.0000x reference)
import jax
import jax.numpy as jnp
from jax.experimental import pallas as pl


def kernel(x, edge_index, edge_attr, batch, emb, e1w, e1b, e2w, e2b, eu1w, eu1b, eu2w, eu2b, nu1w, nu1b, nu2w, nu2b, o1w, o1b, o2w, o2b):
    raise NotImplementedError("write your pallas kernel here")


# speedup vs baseline: 3.1410x; 3.1410x over previous
import functools

import jax
import jax.numpy as jnp
from jax import lax
from jax.experimental import pallas as pl
from jax.experimental.pallas import tpu as pltpu
from jax.experimental.pallas import tpu_sc as plsc

N = 50000
E = 800000
G = 64
NF = 64
EF = 32
GF = 16
NB = 3

NC = 2
NS = 16
TILES = NC * NS
CHUNK = 1000
CPT = E // (TILES * CHUNK)
NPT = N // NS

NBLK = 2000
EBLK = 4000
NGB = N // NBLK
CEF = 16
SCHUNK = 500
SCPT = E // (TILES * SCHUNK)


def _silu(v):
    return v * jax.nn.sigmoid(v)


def _sc_mesh():
    return plsc.VectorSubcoreMesh(core_axis_name="c", subcore_axis_name="s")


def _sc_gather(a, b, row2d, col2d):
    @functools.partial(
        pl.kernel,
        out_type=(jax.ShapeDtypeStruct((E, EF), jnp.float32),
                  jax.ShapeDtypeStruct((E, EF), jnp.float32)),
        mesh=_sc_mesh(),
        compiler_params=pltpu.CompilerParams(use_tc_tiling_on_sc=False),
        scratch_types=[
            pltpu.VMEM((CHUNK,), jnp.int32),
            pltpu.VMEM((CHUNK,), jnp.int32),
            pltpu.VMEM((CHUNK, EF), jnp.float32),
            pltpu.VMEM((CHUNK, EF), jnp.float32),
            pltpu.SemaphoreType.DMA,
            pltpu.SemaphoreType.DMA,
        ],
    )
    def k(a_hbm, b_hbm, row_hbm, col_hbm, ga_hbm, gb_hbm,
          row_v, col_v, bufa, bufb, sema, semb):
        wid = lax.axis_index("s") * NC + lax.axis_index("c")

        @pl.loop(0, CPT)
        def _(c):
            g = wid * CPT + c
            pltpu.sync_copy(row_hbm.at[g], row_v)
            cpa = pltpu.async_copy(a_hbm.at[row_v], bufa, sema)
            pltpu.sync_copy(col_hbm.at[g], col_v)
            cpb = pltpu.async_copy(b_hbm.at[col_v], bufb, semb)
            cpa.wait()
            pltpu.sync_copy(bufa, ga_hbm.at[pl.ds(g * CHUNK, CHUNK)])
            cpb.wait()
            pltpu.sync_copy(bufb, gb_hbm.at[pl.ds(g * CHUNK, CHUNK)])

    return k(a, b, row2d, col2d)


def _sc_scatter(ef, col2d, zeros32):
    @functools.partial(
        pl.kernel,
        out_type=jax.ShapeDtypeStruct((NC, N, EF), jnp.float32),
        mesh=_sc_mesh(),
        compiler_params=pltpu.CompilerParams(use_tc_tiling_on_sc=False),
        scratch_types=[
            pltpu.VMEM((SCHUNK,), jnp.int32),
            pltpu.VMEM((SCHUNK, EF), jnp.float32),
            pltpu.VMEM_SHARED((N, EF), jnp.float32),
        ],
    )
    def k(ef_hbm, col_hbm, z_hbm, out_hbm, col_v, ef_v, acc):
        cid = lax.axis_index("c")
        sid = lax.axis_index("s")
        wid = sid * NC + cid
        pltpu.sync_copy(z_hbm, acc.at[pl.ds(sid * NPT, NPT)])
        plsc.subcore_barrier()

        @pl.loop(0, SCPT)
        def _(c):
            g = wid * SCPT + c
            pltpu.sync_copy(col_hbm.at[g], col_v)
            pltpu.sync_copy(ef_hbm.at[pl.ds(g * SCHUNK, SCHUNK)], ef_v)
            pltpu.sync_copy(ef_v, acc.at[col_v], add=True)

        plsc.subcore_barrier()
        pltpu.sync_copy(acc.at[pl.ds(sid * NPT, NPT)],
                        out_hbm.at[cid, pl.ds(sid * NPT, NPT)])

    return k(ef, col2d, zeros32)


def _sc_count(col2d, ones16, zeros16):
    @functools.partial(
        pl.kernel,
        out_type=jax.ShapeDtypeStruct((NC, N, CEF), jnp.float32),
        mesh=_sc_mesh(),
        compiler_params=pltpu.CompilerParams(use_tc_tiling_on_sc=False),
        scratch_types=[
            pltpu.VMEM((CHUNK,), jnp.int32),
            pltpu.VMEM((CHUNK, CEF), jnp.float32),
            pltpu.VMEM_SHARED((N, CEF), jnp.float32),
        ],
    )
    def k(col_hbm, ones_hbm, z_hbm, out_hbm, col_v, ones_v, acc):
        cid = lax.axis_index("c")
        sid = lax.axis_index("s")
        wid = sid * NC + cid
        pltpu.sync_copy(z_hbm, acc.at[pl.ds(sid * NPT, NPT)])
        pltpu.sync_copy(ones_hbm, ones_v)
        plsc.subcore_barrier()

        @pl.loop(0, CPT)
        def _(c):
            g = wid * CPT + c
            pltpu.sync_copy(col_hbm.at[g], col_v)
            pltpu.sync_copy(ones_v, acc.at[col_v], add=True)

        plsc.subcore_barrier()
        pltpu.sync_copy(acc.at[pl.ds(sid * NPT, NPT)],
                        out_hbm.at[cid, pl.ds(sid * NPT, NPT)])

    return k(col2d, ones16, zeros16)


def _tc_init(x, emb_pad, wr, wc):
    def body(x_ref, emb_ref, wr_ref, wc_ref, h_ref, a_ref, b_ref):
        io = lax.broadcasted_iota(jnp.int32, (NBLK, 128), 1)
        oh = (x_ref[...] == io).astype(jnp.float32)
        h = jnp.dot(oh, emb_ref[...], preferred_element_type=jnp.float32)
        h_ref[...] = h
        a_ref[...] = jnp.dot(h, wr_ref[...], preferred_element_type=jnp.float32)
        b_ref[...] = jnp.dot(h, wc_ref[...], preferred_element_type=jnp.float32)

    full = lambda s: pl.BlockSpec(s, lambda n: (0,) * len(s))
    return pl.pallas_call(
        body,
        grid=(NGB,),
        in_specs=[
            pl.BlockSpec((NBLK, 1), lambda n: (n, 0)),
            full((128, NF)),
            full((NF, EF)),
            full((NF, EF)),
        ],
        out_specs=[
            pl.BlockSpec((NBLK, NF), lambda n: (n, 0)),
            pl.BlockSpec((NBLK, EF), lambda n: (n, 0)),
            pl.BlockSpec((NBLK, EF), lambda n: (n, 0)),
        ],
        out_shape=[
            jax.ShapeDtypeStruct((N, NF), jnp.float32),
            jax.ShapeDtypeStruct((N, EF), jnp.float32),
            jax.ShapeDtypeStruct((N, EF), jnp.float32),
        ],
    )(x, emb_pad, wr, wc)


def _tc_edge(ga, gb, ef_in, we, b1, w2, b2, first, e1w, e1b, e2w, e2b):
    def body(ga_ref, gb_ref, ef_ref, we_ref, b1_ref, w2_ref, b2_ref,
             e1w_ref, e1b_ref, e2w_ref, e2b_ref, out_ref):
        if first:
            ea = ef_ref[...]
            ef = jnp.dot(_silu(ea * e1w_ref[...] + e1b_ref[...]), e2w_ref[...],
                         preferred_element_type=jnp.float32) + e2b_ref[...]
        else:
            ef = ef_ref[...]
        u = (ga_ref[...] + gb_ref[...]
             + jnp.dot(ef, we_ref[...], preferred_element_type=jnp.float32)
             + b1_ref[...])
        out_ref[...] = ef + jnp.dot(_silu(u), w2_ref[...],
                                    preferred_element_type=jnp.float32) + b2_ref[...]

    full = lambda s: pl.BlockSpec(s, lambda e: (0,) * len(s))
    efw = 1 if first else EF
    return pl.pallas_call(
        body,
        grid=(E // EBLK,),
        in_specs=[
            pl.BlockSpec((EBLK, EF), lambda e: (e, 0)),
            pl.BlockSpec((EBLK, EF), lambda e: (e, 0)),
            pl.BlockSpec((EBLK, efw), lambda e: (e, 0)),
            full((EF, EF)),
            full((1, EF)),
            full((EF, EF)),
            full((1, EF)),
            full((1, EF)),
            full((1, EF)),
            full((EF, EF)),
            full((1, EF)),
        ],
        out_specs=pl.BlockSpec((EBLK, EF), lambda e: (e, 0)),
        out_shape=jax.ShapeDtypeStruct((E, EF), jnp.float32),
    )(ga, gb, ef_in, we, b1, w2, b2, e1w, e1b, e2w, e2b)


def _tc_node(h, part, cnt_in, wnh, wna, b1, w2, b2, first, wr, wc):
    has_next = wr is not None

    def body(*refs):
        it = iter(refs)
        h_ref, p_ref, cnt_ref = next(it), next(it), next(it)
        wnh_ref, wna_ref, b1_ref, w2_ref, b2_ref = (next(it) for _ in range(5))
        wr_ref = next(it) if has_next else None
        wc_ref = next(it) if has_next else None
        hout_ref = next(it)
        cntout_ref = next(it) if first else None
        aout_ref = next(it) if has_next else None
        bout_ref = next(it) if has_next else None

        h_v = h_ref[...]
        p = p_ref[0] + p_ref[1]
        if first:
            cnt = cnt_ref[0, :, 0] + cnt_ref[1, :, 0]
            cntout_ref[...] = cnt[:, None]
        else:
            cnt = cnt_ref[...][:, 0]
        r = 1.0 / jnp.maximum(cnt, 1.0)
        agg = p * r[:, None]
        pre = (jnp.dot(h_v, wnh_ref[...], preferred_element_type=jnp.float32)
               + jnp.dot(agg, wna_ref[...], preferred_element_type=jnp.float32)
               + b1_ref[...])
        hn = h_v + jnp.dot(_silu(pre), w2_ref[...],
                           preferred_element_type=jnp.float32) + b2_ref[...]
        hout_ref[...] = hn
        if has_next:
            aout_ref[...] = jnp.dot(hn, wr_ref[...],
                                    preferred_element_type=jnp.float32)
            bout_ref[...] = jnp.dot(hn, wc_ref[...],
                                    preferred_element_type=jnp.float32)

    full = lambda s: pl.BlockSpec(s, lambda n: (0,) * len(s))
    in_specs = [
        pl.BlockSpec((NBLK, NF), lambda n: (n, 0)),
        pl.BlockSpec((NC, NBLK, EF), lambda n: (0, n, 0)),
        (pl.BlockSpec((NC, NBLK, CEF), lambda n: (0, n, 0)) if first
         else pl.BlockSpec((NBLK, 1), lambda n: (n, 0))),
        full((NF, NF)),
        full((EF, NF)),
        full((1, NF)),
        full((NF, NF)),
        full((1, NF)),
    ]
    inputs = [h, part, cnt_in, wnh, wna, b1, w2, b2]
    if has_next:
        in_specs += [full((NF, EF)), full((NF, EF))]
        inputs += [wr, wc]
    out_specs = [pl.BlockSpec((NBLK, NF), lambda n: (n, 0))]
    out_shape = [jax.ShapeDtypeStruct((N, NF), jnp.float32)]
    if first:
        out_specs.append(pl.BlockSpec((NBLK, 1), lambda n: (n, 0)))
        out_shape.append(jax.ShapeDtypeStruct((N, 1), jnp.float32))
    if has_next:
        out_specs += [pl.BlockSpec((NBLK, EF), lambda n: (n, 0))] * 2
        out_shape += [jax.ShapeDtypeStruct((N, EF), jnp.float32)] * 2
    return pl.pallas_call(
        body,
        grid=(NGB,),
        in_specs=in_specs,
        out_specs=out_specs,
        out_shape=out_shape,
    )(*inputs)


def _tc_pool(h, batch3, o1w, o1b, o2w, o2b):
    def body(b_ref, h_ref, o1w_ref, o1b_ref, o2w_ref, o2b_ref, out_ref,
             gsum, gcnt):
        nb = pl.program_id(0)

        @pl.when(nb == 0)
        def _():
            gsum[...] = jnp.zeros_like(gsum)
            gcnt[...] = jnp.zeros_like(gcnt)

        bt = b_ref[0, 0, :]
        oh = (bt[:, None] == lax.broadcasted_iota(jnp.int32, (NBLK, G), 1)
              ).astype(jnp.float32)
        gsum[...] += lax.dot_general(oh, h_ref[...], (((0,), (0,)), ((), ())),
                                     preferred_element_type=jnp.float32)
        gcnt[...] += jnp.sum(oh, axis=0)[:, None]

        @pl.when(nb == NGB - 1)
        def _():
            gm = gsum[...] / jnp.maximum(gcnt[...], 1.0)
            z = _silu(jnp.dot(gm, o1w_ref[...],
                              preferred_element_type=jnp.float32) + o1b_ref[...])
            out_ref[...] = jnp.dot(z, o2w_ref[...],
                                   preferred_element_type=jnp.float32) + o2b_ref[...]

    full = lambda s: pl.BlockSpec(s, lambda n: (0,) * len(s))
    return pl.pallas_call(
        body,
        grid=(NGB,),
        in_specs=[
            pl.BlockSpec((1, 1, NBLK), lambda n: (n, 0, 0)),
            pl.BlockSpec((NBLK, NF), lambda n: (n, 0)),
            full((NF, GF)),
            full((1, GF)),
            full((GF, 1)),
            full((1, 1)),
        ],
        out_specs=pl.BlockSpec((G, 1), lambda n: (0, 0)),
        out_shape=jax.ShapeDtypeStruct((G, 1), jnp.float32),
        scratch_shapes=[
            pltpu.VMEM((G, NF), jnp.float32),
            pltpu.VMEM((G, 1), jnp.float32),
        ],
    )(batch3, h, o1w, o1b, o2w, o2b)


def kernel(x, edge_index, edge_attr, batch, emb, e1w, e1b, e2w, e2b,
           eu1w, eu1b, eu2w, eu2b, nu1w, nu1b, nu2w, nu2b,
           o1w, o1b, o2w, o2b):
    f32 = jnp.float32
    row2d = edge_index[0].astype(jnp.int32).reshape(E // CHUNK, CHUNK)
    col2d = edge_index[1].astype(jnp.int32).reshape(E // CHUNK, CHUNK)
    col2ds = edge_index[1].astype(jnp.int32).reshape(E // SCHUNK, SCHUNK)
    emb_pad = jnp.zeros((128, NF), f32).at[:100].set(emb.astype(f32))
    batch3 = batch.astype(jnp.int32).reshape(NGB, 1, NBLK)

    zeros32 = jnp.zeros((NPT, EF), f32)
    zeros16 = jnp.zeros((NPT, CEF), f32)
    ones16 = jnp.ones((CHUNK, CEF), f32)

    wr = eu1w[:, :NF, :]
    wc = eu1w[:, NF:2 * NF, :]
    we = eu1w[:, 2 * NF:, :]
    wnh = nu1w[:, :NF, :]
    wna = nu1w[:, NF:, :]

    r2 = lambda v: v.reshape(1, -1)

    cntp = _sc_count(col2d, ones16, zeros16)

    h, a, b = _tc_init(x.astype(jnp.int32), emb_pad, wr[0], wc[0])
    ef = edge_attr.astype(f32)
    cnt = None
    for i in range(NB):
        ga, gb = _sc_gather(a, b, row2d, col2d)
        ef = _tc_edge(ga, gb, ef, we[i], r2(eu1b[i]), eu2w[i], r2(eu2b[i]),
                      first=(i == 0), e1w=e1w, e1b=r2(e1b), e2w=e2w,
                      e2b=r2(e2b))
        part = _sc_scatter(ef, col2ds, zeros32)
        has_next = i + 1 < NB
        res = _tc_node(h, part, cntp if i == 0 else cnt,
                       wnh[i], wna[i], r2(nu1b[i]), nu2w[i], r2(nu2b[i]),
                       first=(i == 0),
                       wr=wr[i + 1] if has_next else None,
                       wc=wc[i + 1] if has_next else None)
        if i == 0:
            h, cnt, a, b = res
        elif has_next:
            h, a, b = res
        else:
            h = res[0] if isinstance(res, (tuple, list)) else res

    return _tc_pool(h, batch3, o1w, r2(o1b), o2w, r2(o2b))


# speedup vs baseline: 7.6306x; 2.4294x over previous
import functools

import jax
import jax.numpy as jnp
from jax import lax
from jax.experimental import pallas as pl
from jax.experimental.pallas import tpu as pltpu
from jax.experimental.pallas import tpu_sc as plsc

N = 50000
E = 800000
G = 64
NF = 64
EF = 32
GF = 16
NB = 3

NC = 2
NS = 16
TILES = NC * NS
CHUNK = 1000
CPT = E // (TILES * CHUNK)
NPT = N // NS

NBLK = 2000
EBLK = 4000
NGB = N // NBLK
CEF = 16
SCHUNK = 250
SCPT = E // (TILES * SCHUNK)
GCH = 500
GCPT = E // (TILES * GCH)

EP = E // 4
BP = EBLK // 4


def _silu(v):
    return v * jax.nn.sigmoid(v)


def _sc_mesh():
    return plsc.VectorSubcoreMesh(core_axis_name="c", subcore_axis_name="s")


def _sc_gather(a, b, row2d, col2d):
    @functools.partial(
        pl.kernel,
        out_type=(jax.ShapeDtypeStruct((E, EF), jnp.float32),
                  jax.ShapeDtypeStruct((E, EF), jnp.float32)),
        mesh=_sc_mesh(),
        compiler_params=pltpu.CompilerParams(use_tc_tiling_on_sc=False),
        scratch_types=[
            [pltpu.VMEM((GCH,), jnp.int32)] * 2,
            [pltpu.VMEM((GCH,), jnp.int32)] * 2,
            [pltpu.VMEM((GCH, EF), jnp.float32)] * 2,
            [pltpu.VMEM((GCH, EF), jnp.float32)] * 2,
            [pltpu.SemaphoreType.DMA] * 2,
            [pltpu.SemaphoreType.DMA] * 2,
        ],
    )
    def k(a_hbm, b_hbm, row_hbm, col_hbm, ga_hbm, gb_hbm,
          rv, cv, bufa, bufb, sema, semb):
        wid = lax.axis_index("s") * NC + lax.axis_index("c")

        def start(c, b):
            g = wid * GCPT + c
            pltpu.sync_copy(row_hbm.at[g], rv[b])
            pltpu.async_copy(a_hbm.at[rv[b]], bufa[b], sema[b])
            pltpu.sync_copy(col_hbm.at[g], cv[b])
            pltpu.async_copy(b_hbm.at[cv[b]], bufb[b], semb[b])

        for b in (0, 1):
            start(b, b)

        @pl.loop(0, GCPT // 2)
        def _(p):
            for b in (0, 1):
                c = p * 2 + b
                base = (wid * GCPT + c) * GCH
                pltpu.make_async_copy(a_hbm.at[rv[b]], bufa[b], sema[b]).wait()
                pltpu.sync_copy(bufa[b], ga_hbm.at[pl.ds(base, GCH)])
                pltpu.make_async_copy(b_hbm.at[cv[b]], bufb[b], semb[b]).wait()
                pltpu.sync_copy(bufb[b], gb_hbm.at[pl.ds(base, GCH)])

                @pl.when(p < GCPT // 2 - 1)
                def _():
                    start(c + 2, b)

    ga, gb = k(a, b, row2d, col2d)
    return ga.reshape(EP, 128), gb.reshape(EP, 128)


def _sc_scatter(ef, col2d, zeros32):
    @functools.partial(
        pl.kernel,
        out_type=jax.ShapeDtypeStruct((NC, N, EF), jnp.float32),
        mesh=_sc_mesh(),
        compiler_params=pltpu.CompilerParams(use_tc_tiling_on_sc=False),
        scratch_types=[
            [pltpu.VMEM((SCHUNK,), jnp.int32)] * 2,
            [pltpu.VMEM((SCHUNK, EF), jnp.float32)] * 2,
            pltpu.VMEM_SHARED((N, EF), jnp.float32),
            [pltpu.SemaphoreType.DMA] * 2,
            [pltpu.SemaphoreType.DMA] * 2,
        ],
    )
    def k(ef_hbm, col_hbm, z_hbm, out_hbm, col_v, ef_v, acc, sc, se):
        cid = lax.axis_index("c")
        sid = lax.axis_index("s")
        wid = sid * NC + cid
        pltpu.sync_copy(z_hbm, acc.at[pl.ds(sid * NPT, NPT)])
        plsc.subcore_barrier()

        def start(c, b):
            g = wid * SCPT + c
            pltpu.async_copy(col_hbm.at[g], col_v[b], sc[b])
            pltpu.async_copy(ef_hbm.at[pl.ds(g * SCHUNK, SCHUNK)], ef_v[b],
                             se[b])

        def finish(c, b):
            g = wid * SCPT + c
            pltpu.make_async_copy(col_hbm.at[g], col_v[b], sc[b]).wait()
            pltpu.make_async_copy(
                ef_hbm.at[pl.ds(g * SCHUNK, SCHUNK)], ef_v[b], se[b]).wait()
            pltpu.sync_copy(ef_v[b], acc.at[col_v[b]], add=True)

        for b in (0, 1):
            start(b, b)

        @pl.loop(0, SCPT // 2)
        def _(p):
            for b in (0, 1):
                c = p * 2 + b
                finish(c, b)

                @pl.when(c + 2 < SCPT)
                def _():
                    start(c + 2, b)

        if SCPT % 2 == 1:
            finish(SCPT - 1, 0)

        plsc.subcore_barrier()
        pltpu.sync_copy(acc.at[pl.ds(sid * NPT, NPT)],
                        out_hbm.at[cid, pl.ds(sid * NPT, NPT)])

    return k(ef.reshape(E, EF), col2d, zeros32)


def _sc_count(col2d, ones16, zeros16):
    @functools.partial(
        pl.kernel,
        out_type=jax.ShapeDtypeStruct((NC, N, CEF), jnp.float32),
        mesh=_sc_mesh(),
        compiler_params=pltpu.CompilerParams(use_tc_tiling_on_sc=False),
        scratch_types=[
            pltpu.VMEM((CHUNK,), jnp.int32),
            pltpu.VMEM((CHUNK, CEF), jnp.float32),
            pltpu.VMEM_SHARED((N, CEF), jnp.float32),
        ],
    )
    def k(col_hbm, ones_hbm, z_hbm, out_hbm, col_v, ones_v, acc):
        cid = lax.axis_index("c")
        sid = lax.axis_index("s")
        wid = sid * NC + cid
        pltpu.sync_copy(z_hbm, acc.at[pl.ds(sid * NPT, NPT)])
        pltpu.sync_copy(ones_hbm, ones_v)
        plsc.subcore_barrier()

        @pl.loop(0, CPT)
        def _(c):
            g = wid * CPT + c
            pltpu.sync_copy(col_hbm.at[g], col_v)
            pltpu.sync_copy(ones_v, acc.at[col_v], add=True)

        plsc.subcore_barrier()
        pltpu.sync_copy(acc.at[pl.ds(sid * NPT, NPT)],
                        out_hbm.at[cid, pl.ds(sid * NPT, NPT)])

    return k(col2d, ones16, zeros16)


def _tc_init(x, emb_pad, wr, wc):
    def body(x_ref, emb_ref, wr_ref, wc_ref, h_ref, a_ref, b_ref):
        io = lax.broadcasted_iota(jnp.int32, (NBLK, 128), 1)
        oh = (x_ref[...] == io).astype(jnp.float32)
        h = jnp.dot(oh, emb_ref[...], preferred_element_type=jnp.float32)
        h_ref[...] = h
        a_ref[...] = jnp.dot(h, wr_ref[...], preferred_element_type=jnp.float32)
        b_ref[...] = jnp.dot(h, wc_ref[...], preferred_element_type=jnp.float32)

    full = lambda s: pl.BlockSpec(s, lambda n: (0,) * len(s))
    return pl.pallas_call(
        body,
        grid=(NGB,),
        in_specs=[
            pl.BlockSpec((NBLK, 1), lambda n: (n, 0)),
            full((128, NF)),
            full((NF, EF)),
            full((NF, EF)),
        ],
        out_specs=[
            pl.BlockSpec((NBLK, NF), lambda n: (n, 0)),
            pl.BlockSpec((NBLK, EF), lambda n: (n, 0)),
            pl.BlockSpec((NBLK, EF), lambda n: (n, 0)),
        ],
        out_shape=[
            jax.ShapeDtypeStruct((N, NF), jnp.float32),
            jax.ShapeDtypeStruct((N, EF), jnp.float32),
            jax.ShapeDtypeStruct((N, EF), jnp.float32),
        ],
    )(x, emb_pad, wr, wc)


def _tc_edge(ga, gb, ef_in, we, b1, w2, b2, first, e1w, e1b, e2w, e2b):
    def body(ga_ref, gb_ref, ef_ref, we_ref, b1_ref, w2_ref, b2_ref,
             e1w_ref, e1b_ref, e2w_ref, e2b_ref, out_ref):
        if first:
            ea = ef_ref[...]
            ef = jnp.dot(_silu(ea * e1w_ref[...] + e1b_ref[...]), e2w_ref[...],
                         preferred_element_type=jnp.float32) + e2b_ref[...]
        else:
            ef = ef_ref[...]
        u = (ga_ref[...] + gb_ref[...]
             + jnp.dot(ef, we_ref[...], preferred_element_type=jnp.float32)
             + b1_ref[...])
        out_ref[...] = ef + jnp.dot(_silu(u), w2_ref[...],
                                    preferred_element_type=jnp.float32) + b2_ref[...]

    full = lambda s: pl.BlockSpec(s, lambda e: (0,) * len(s))
    return pl.pallas_call(
        body,
        grid=(EP // BP,),
        in_specs=[
            pl.BlockSpec((BP, 128), lambda e: (e, 0)),
            pl.BlockSpec((BP, 128), lambda e: (e, 0)),
            pl.BlockSpec((BP, 128), lambda e: (e, 0)),
            full((128, 128)),
            full((1, 128)),
            full((128, 128)),
            full((1, 128)),
            full((1, 128)),
            full((1, 128)),
            full((128, 128)),
            full((1, 128)),
        ],
        out_specs=pl.BlockSpec((BP, 128), lambda e: (e, 0)),
        out_shape=jax.ShapeDtypeStruct((EP, 128), jnp.float32),
    )(ga, gb, ef_in, we, b1, w2, b2, e1w, e1b, e2w, e2b)


def _tc_node(h, part, cnt_in, wnh, wna, b1, w2, b2, first, wr, wc):
    has_next = wr is not None

    def body(*refs):
        it = iter(refs)
        h_ref, p_ref, cnt_ref = next(it), next(it), next(it)
        wnh_ref, wna_ref, b1_ref, w2_ref, b2_ref = (next(it) for _ in range(5))
        wr_ref = next(it) if has_next else None
        wc_ref = next(it) if has_next else None
        hout_ref = next(it)
        cntout_ref = next(it) if first else None
        aout_ref = next(it) if has_next else None
        bout_ref = next(it) if has_next else None

        h_v = h_ref[...]
        p = p_ref[0] + p_ref[1]
        if first:
            cnt = cnt_ref[0, :, 0] + cnt_ref[1, :, 0]
            cntout_ref[...] = cnt[:, None]
        else:
            cnt = cnt_ref[...][:, 0]
        r = 1.0 / jnp.maximum(cnt, 1.0)
        agg = p * r[:, None]
        pre = (jnp.dot(h_v, wnh_ref[...], preferred_element_type=jnp.float32)
               + jnp.dot(agg, wna_ref[...], preferred_element_type=jnp.float32)
               + b1_ref[...])
        hn = h_v + jnp.dot(_silu(pre), w2_ref[...],
                           preferred_element_type=jnp.float32) + b2_ref[...]
        hout_ref[...] = hn
        if has_next:
            aout_ref[...] = jnp.dot(hn, wr_ref[...],
                                    preferred_element_type=jnp.float32)
            bout_ref[...] = jnp.dot(hn, wc_ref[...],
                                    preferred_element_type=jnp.float32)

    full = lambda s: pl.BlockSpec(s, lambda n: (0,) * len(s))
    in_specs = [
        pl.BlockSpec((NBLK, NF), lambda n: (n, 0)),
        pl.BlockSpec((NC, NBLK, EF), lambda n: (0, n, 0)),
        (pl.BlockSpec((NC, NBLK, CEF), lambda n: (0, n, 0)) if first
         else pl.BlockSpec((NBLK, 1), lambda n: (n, 0))),
        full((NF, NF)),
        full((EF, NF)),
        full((1, NF)),
        full((NF, NF)),
        full((1, NF)),
    ]
    inputs = [h, part, cnt_in, wnh, wna, b1, w2, b2]
    if has_next:
        in_specs += [full((NF, EF)), full((NF, EF))]
        inputs += [wr, wc]
    out_specs = [pl.BlockSpec((NBLK, NF), lambda n: (n, 0))]
    out_shape = [jax.ShapeDtypeStruct((N, NF), jnp.float32)]
    if first:
        out_specs.append(pl.BlockSpec((NBLK, 1), lambda n: (n, 0)))
        out_shape.append(jax.ShapeDtypeStruct((N, 1), jnp.float32))
    if has_next:
        out_specs += [pl.BlockSpec((NBLK, EF), lambda n: (n, 0))] * 2
        out_shape += [jax.ShapeDtypeStruct((N, EF), jnp.float32)] * 2
    return pl.pallas_call(
        body,
        grid=(NGB,),
        in_specs=in_specs,
        out_specs=out_specs,
        out_shape=out_shape,
    )(*inputs)


def _tc_pool(h, batch3, o1w, o1b, o2w, o2b):
    def body(b_ref, h_ref, o1w_ref, o1b_ref, o2w_ref, o2b_ref, out_ref,
             gsum, gcnt):
        nb = pl.program_id(0)

        @pl.when(nb == 0)
        def _():
            gsum[...] = jnp.zeros_like(gsum)
            gcnt[...] = jnp.zeros_like(gcnt)

        bt = b_ref[0, 0, :]
        oh = (bt[:, None] == lax.broadcasted_iota(jnp.int32, (NBLK, G), 1)
              ).astype(jnp.float32)
        gsum[...] += lax.dot_general(oh, h_ref[...], (((0,), (0,)), ((), ())),
                                     preferred_element_type=jnp.float32)
        gcnt[...] += jnp.sum(oh, axis=0)[:, None]

        @pl.when(nb == NGB - 1)
        def _():
            gm = gsum[...] / jnp.maximum(gcnt[...], 1.0)
            z = _silu(jnp.dot(gm, o1w_ref[...],
                              preferred_element_type=jnp.float32) + o1b_ref[...])
            out_ref[...] = jnp.dot(z, o2w_ref[...],
                                   preferred_element_type=jnp.float32) + o2b_ref[...]

    full = lambda s: pl.BlockSpec(s, lambda n: (0,) * len(s))
    return pl.pallas_call(
        body,
        grid=(NGB,),
        in_specs=[
            pl.BlockSpec((1, 1, NBLK), lambda n: (n, 0, 0)),
            pl.BlockSpec((NBLK, NF), lambda n: (n, 0)),
            full((NF, GF)),
            full((1, GF)),
            full((GF, 1)),
            full((1, 1)),
        ],
        out_specs=pl.BlockSpec((G, 1), lambda n: (0, 0)),
        out_shape=jax.ShapeDtypeStruct((G, 1), jnp.float32),
        scratch_shapes=[
            pltpu.VMEM((G, NF), jnp.float32),
            pltpu.VMEM((G, 1), jnp.float32),
        ],
    )(batch3, h, o1w, o1b, o2w, o2b)


def kernel(x, edge_index, edge_attr, batch, emb, e1w, e1b, e2w, e2b,
           eu1w, eu1b, eu2w, eu2b, nu1w, nu1b, nu2w, nu2b,
           o1w, o1b, o2w, o2b):
    f32 = jnp.float32
    row2d = edge_index[0].astype(jnp.int32).reshape(E // GCH, GCH)
    col2d = edge_index[1].astype(jnp.int32).reshape(E // GCH, GCH)
    col2dc = edge_index[1].astype(jnp.int32).reshape(E // CHUNK, CHUNK)
    col2ds = edge_index[1].astype(jnp.int32).reshape(E // SCHUNK, SCHUNK)
    ea_p = jnp.repeat(edge_attr.astype(f32).reshape(-1), EF).reshape(EP, 128)
    eye4 = jnp.eye(4, dtype=f32)
    bd = lambda w: jnp.kron(eye4, w.astype(f32))
    t4 = lambda v: jnp.tile(v.astype(f32).reshape(1, -1), (1, 4))
    emb_pad = jnp.zeros((128, NF), f32).at[:100].set(emb.astype(f32))
    batch3 = batch.astype(jnp.int32).reshape(NGB, 1, NBLK)

    zeros32 = jnp.zeros((NPT, EF), f32)
    zeros16 = jnp.zeros((NPT, CEF), f32)
    ones16 = jnp.ones((CHUNK, CEF), f32)

    wr = eu1w[:, :NF, :]
    wc = eu1w[:, NF:2 * NF, :]
    we = eu1w[:, 2 * NF:, :]
    wnh = nu1w[:, :NF, :]
    wna = nu1w[:, NF:, :]

    r2 = lambda v: v.reshape(1, -1)

    cntp = _sc_count(col2dc, ones16, zeros16)

    h, a, b = _tc_init(x.astype(jnp.int32), emb_pad, wr[0], wc[0])
    ef = ea_p
    cnt = None
    for i in range(NB):
        ga, gb = _sc_gather(a, b, row2d, col2d)
        ef = _tc_edge(ga, gb, ef, bd(we[i]), t4(eu1b[i]), bd(eu2w[i]),
                      t4(eu2b[i]), first=(i == 0), e1w=t4(e1w), e1b=t4(e1b),
                      e2w=bd(e2w), e2b=t4(e2b))
        part = _sc_scatter(ef, col2ds, zeros32)
        has_next = i + 1 < NB
        res = _tc_node(h, part, cntp if i == 0 else cnt,
                       wnh[i], wna[i], r2(nu1b[i]), nu2w[i], r2(nu2b[i]),
                       first=(i == 0),
                       wr=wr[i + 1] if has_next else None,
                       wc=wc[i + 1] if has_next else None)
        if i == 0:
            h, cnt, a, b = res
        elif has_next:
            h, a, b = res
        else:
            h = res[0] if isinstance(res, (tuple, list)) else res

    return _tc_pool(h, batch3, o1w, r2(o1b), o2w, r2(o2b))
